# initial kernel scaffold (unmeasured)
import jax
import jax.numpy as jnp
from jax import lax
from jax.experimental import pallas as pl
from jax.experimental.pallas import tpu as pltpu

N_DEV = 4


def kernel(x, w_mat, scale_x, scale_w):
    m_per, k = x.shape
    _, n_per = w_mat.shape
    half = m_per // 2

    def body(x_ref, w_ref, sx_ref, sw_ref, out_ref,
             recv_l, recv_r, recv_d, send_sems, recv_sems):
        my = lax.axis_index("i")
        left = lax.rem(my + (N_DEV - 1), N_DEV)
        right = lax.rem(my + 1, N_DEV)

        barrier_sem = pltpu.get_barrier_semaphore()
        pl.semaphore_signal(barrier_sem, inc=1, device_id=(left,),
                            device_id_type=pl.DeviceIdType.MESH)
        pl.semaphore_signal(barrier_sem, inc=1, device_id=(right,),
                            device_id_type=pl.DeviceIdType.MESH)
        pl.semaphore_wait(barrier_sem, 2)

        p1r = pltpu.make_async_remote_copy(
            src_ref=x_ref, dst_ref=recv_l,
            send_sem=send_sems.at[0], recv_sem=recv_sems.at[0],
            device_id=(right,), device_id_type=pl.DeviceIdType.MESH)
        p1l = pltpu.make_async_remote_copy(
            src_ref=x_ref, dst_ref=recv_r,
            send_sem=send_sems.at[1], recv_sem=recv_sems.at[1],
            device_id=(left,), device_id_type=pl.DeviceIdType.MESH)
        p1r.start()
        p1l.start()

        scale = sx_ref[0] * sw_ref[0]
        w = w_ref[...].astype(jnp.bfloat16)

        def block(a):
            acc = lax.dot_general(
                a.astype(jnp.bfloat16), w,
                (((1,), (0,)), ((), ())),
                preferred_element_type=jnp.float32)
            return jnp.maximum(acc * scale, 0.0)

        out_ref[pl.ds(my * m_per, m_per), :] = block(x_ref[...])

        p1r.wait_recv()
        p1l.wait_recv()

        p2r = pltpu.make_async_remote_copy(
            src_ref=recv_l.at[pl.ds(0, half)],
            dst_ref=recv_d.at[pl.ds(0, half)],
            send_sem=send_sems.at[2], recv_sem=recv_sems.at[2],
            device_id=(right,), device_id_type=pl.DeviceIdType.MESH)
        p2l = pltpu.make_async_remote_copy(
            src_ref=recv_r.at[pl.ds(half, half)],
            dst_ref=recv_d.at[pl.ds(half, half)],
            send_sem=send_sems.at[3], recv_sem=recv_sems.at[3],
            device_id=(left,), device_id_type=pl.DeviceIdType.MESH)
        p2r.start()
        p2l.start()

        out_ref[pl.ds(left * m_per, m_per), :] = block(recv_l[...])
        out_ref[pl.ds(right * m_per, m_per), :] = block(recv_r[...])

        p2r.wait_recv()
        p2l.wait_recv()
        diag = lax.rem(my + 2, N_DEV)
        out_ref[pl.ds(diag * m_per, m_per), :] = block(recv_d[...])

        p1r.wait_send()
        p1l.wait_send()
        p2r.wait_send()
        p2l.wait_send()

    return pl.pallas_call(
        body,
        out_shape=jax.ShapeDtypeStruct((N_DEV * m_per, n_per), jnp.float32),
        in_specs=[
            pl.BlockSpec(memory_space=pltpu.VMEM),
            pl.BlockSpec(memory_space=pltpu.VMEM),
            pl.BlockSpec(memory_space=pltpu.SMEM),
            pl.BlockSpec(memory_space=pltpu.SMEM),
        ],
        out_specs=pl.BlockSpec(memory_space=pltpu.VMEM),
        scratch_shapes=[
            pltpu.VMEM((m_per, k), x.dtype),
            pltpu.VMEM((m_per, k), x.dtype),
            pltpu.VMEM((m_per, k), x.dtype),
            pltpu.SemaphoreType.DMA((4,)),
            pltpu.SemaphoreType.DMA((4,)),
        ],
        compiler_params=pltpu.CompilerParams(collective_id=0),
    )(x, w_mat, scale_x, scale_w)


# baseline (device time: 127322 ns/iter reference)
import jax
import jax.numpy as jnp
from jax import lax
from jax.experimental import pallas as pl
from jax.experimental.pallas import tpu as pltpu

N_DEV = 4


def kernel(x, w_mat, scale_x, scale_w):
    m_per, k = x.shape
    _, n_per = w_mat.shape
    half = m_per // 2

    x8 = x.astype(jnp.float8_e4m3fn)
    w8 = w_mat.astype(jnp.float8_e5m2)

    def body(x_ref, w_ref, sx_ref, sw_ref, out_hbm,
             recv_l, recv_r, recv_d, out_vmem,
             send_sems, recv_sems, copy_sems):
        my = lax.axis_index("i")
        left = lax.rem(my + (N_DEV - 1), N_DEV)
        right = lax.rem(my + 1, N_DEV)

        barrier_sem = pltpu.get_barrier_semaphore()
        pl.semaphore_signal(barrier_sem, inc=1, device_id=(left,),
                            device_id_type=pl.DeviceIdType.MESH)
        pl.semaphore_signal(barrier_sem, inc=1, device_id=(right,),
                            device_id_type=pl.DeviceIdType.MESH)
        pl.semaphore_wait(barrier_sem, 2)

        p1r = pltpu.make_async_remote_copy(
            src_ref=x_ref, dst_ref=recv_l,
            send_sem=send_sems.at[0], recv_sem=recv_sems.at[0],
            device_id=(right,), device_id_type=pl.DeviceIdType.MESH)
        p1l = pltpu.make_async_remote_copy(
            src_ref=x_ref, dst_ref=recv_r,
            send_sem=send_sems.at[1], recv_sem=recv_sems.at[1],
            device_id=(left,), device_id_type=pl.DeviceIdType.MESH)
        p1r.start()
        p1l.start()

        scale = sx_ref[0] * sw_ref[0]

        def block(a_ref, slot):
            acc = lax.dot_general(
                a_ref[...], w_ref[...],
                (((1,), (0,)), ((), ())),
                preferred_element_type=jnp.float32)
            out_vmem[slot] = jnp.maximum(acc * scale, 0.0)

        def store(slot, origin):
            cp = pltpu.make_async_copy(
                out_vmem.at[slot],
                out_hbm.at[pl.ds(origin * m_per, m_per)],
                copy_sems.at[slot])
            cp.start()
            return cp

        block(x_ref, 0)
        cp0 = store(0, my)

        p1r.wait_recv()
        p1l.wait_recv()

        p2r = pltpu.make_async_remote_copy(
            src_ref=recv_l.at[pl.ds(0, half)],
            dst_ref=recv_d.at[pl.ds(0, half)],
            send_sem=send_sems.at[2], recv_sem=recv_sems.at[2],
            device_id=(right,), device_id_type=pl.DeviceIdType.MESH)
        p2l = pltpu.make_async_remote_copy(
            src_ref=recv_r.at[pl.ds(half, half)],
            dst_ref=recv_d.at[pl.ds(half, half)],
            send_sem=send_sems.at[3], recv_sem=recv_sems.at[3],
            device_id=(left,), device_id_type=pl.DeviceIdType.MESH)
        p2r.start()
        p2l.start()

        block(recv_l, 1)
        cp1 = store(1, left)

        cp0.wait()
        block(recv_r, 0)
        cp2 = store(0, right)

        p2r.wait_recv()
        p2l.wait_recv()
        diag = lax.rem(my + 2, N_DEV)
        cp1.wait()
        block(recv_d, 1)
        cp3 = store(1, diag)

        cp2.wait()
        cp3.wait()
        p1r.wait_send()
        p1l.wait_send()
        p2r.wait_send()
        p2l.wait_send()

    return pl.pallas_call(
        body,
        out_shape=jax.ShapeDtypeStruct((N_DEV * m_per, n_per), jnp.float32),
        in_specs=[
            pl.BlockSpec(memory_space=pltpu.VMEM),
            pl.BlockSpec(memory_space=pltpu.VMEM),
            pl.BlockSpec(memory_space=pltpu.SMEM),
            pl.BlockSpec(memory_space=pltpu.SMEM),
        ],
        out_specs=pl.BlockSpec(memory_space=pl.ANY),
        scratch_shapes=[
            pltpu.VMEM((m_per, k), jnp.float8_e4m3fn),
            pltpu.VMEM((m_per, k), jnp.float8_e4m3fn),
            pltpu.VMEM((m_per, k), jnp.float8_e4m3fn),
            pltpu.VMEM((2, m_per, n_per), jnp.float32),
            pltpu.SemaphoreType.DMA((4,)),
            pltpu.SemaphoreType.DMA((4,)),
            pltpu.SemaphoreType.DMA((2,)),
        ],
        compiler_params=pltpu.CompilerParams(collective_id=0),
    )(x8, w8, scale_x, scale_w)
